# baseline (speedup 1.0000x reference)
import functools

import jax
import jax.numpy as jnp
from jax import lax
from jax.experimental import pallas as pl
from jax.experimental.pallas import tpu as pltpu
from jax.experimental.pallas import tpu_sc as plsc

B = 16384
EMBED = 16
NC = 2
NS = 16
NW = NC * NS
BPW = B // NW
CHUNK = 128
NCHUNK = BPW // CHUNK
NGRP = CHUNK // 16


def _sc_gather_partial(uemb2, uidx, femb2, fidx, ubias, fbias):
  mesh = plsc.VectorSubcoreMesh(core_axis_name="c", subcore_axis_name="s")

  @functools.partial(
      pl.kernel,
      mesh=mesh,
      compiler_params=pltpu.CompilerParams(needs_layout_passes=False),
      out_type=(
          jax.ShapeDtypeStruct((NW * EMBED,), jnp.float32),
          jax.ShapeDtypeStruct((B,), jnp.float32),
      ),
      scratch_types=[
          pltpu.VMEM((NCHUNK, CHUNK), jnp.int32),
          pltpu.VMEM((NCHUNK, CHUNK), jnp.int32),
          pltpu.VMEM((NCHUNK, CHUNK), jnp.int32),
          pltpu.VMEM((NCHUNK, CHUNK), jnp.int32),
          pltpu.VMEM((CHUNK, 128), jnp.float32),
          pltpu.VMEM((CHUNK, 128), jnp.float32),
          pltpu.VMEM((BPW,), jnp.float32),
          pltpu.VMEM((BPW,), jnp.float32),
          pltpu.VMEM((BPW,), jnp.float32),
          pltpu.VMEM((EMBED,), jnp.float32),
          pltpu.SemaphoreType.DMA,
          pltpu.SemaphoreType.DMA,
      ],
  )
  def k(uemb_h, uidx_h, femb_h, fidx_h, ub_h, fb_h,
        part_h, bsum_h,
        uidx_v, fidx_v, ublk_v, fblk_v, urows_v, frows_v,
        ubv, fbv, bs_v, acc_v, sem, bsem):
    wid = lax.axis_index("s") * NC + lax.axis_index("c")
    base = wid * BPW
    pltpu.sync_copy(uidx_h.at[wid], uidx_v)
    pltpu.sync_copy(fidx_h.at[wid], fidx_v)

    for c in range(NCHUNK):
      for g in range(NGRP):
        sl = pl.ds(g * 16, 16)
        ublk_v[c, sl] = uidx_v[c, sl] >> 3
        fblk_v[c, sl] = fidx_v[c, sl] >> 3

    bias_copies = []
    for c in range(NCHUNK):
      sl = pl.ds(c * CHUNK, CHUNK)
      bias_copies.append(pltpu.async_copy(ub_h.at[uidx_v.at[c]], ubv.at[sl], bsem))
      bias_copies.append(pltpu.async_copy(fb_h.at[fidx_v.at[c]], fbv.at[sl], bsem))

    lanes = lax.iota(jnp.int32, 16)
    zero = jnp.zeros((EMBED,), jnp.float32)
    accs = [zero, zero, zero, zero]
    for c in range(NCHUNK):
      cu = pltpu.async_copy(uemb_h.at[ublk_v.at[c]], urows_v, sem)
      cf = pltpu.async_copy(femb_h.at[fblk_v.at[c]], frows_v, sem)
      cu.wait()
      cf.wait()
      for g in range(NGRP):
        sl = pl.ds(g * 16, 16)
        uidx16 = uidx_v[c, sl]
        fidx16 = fidx_v[c, sl]
        ucol0 = (uidx16 & 7) * 16
        fcol0 = (fidx16 & 7) * 16
        rows = lanes + (g * 16)
        for e in range(EMBED):
          u_e = plsc.load_gather(urows_v, [rows, ucol0 + e])
          f_e = plsc.load_gather(frows_v, [rows, fcol0 + e])
          accs[e % 4] = accs[e % 4] + u_e * f_e

    for cpy in bias_copies:
      cpy.wait()

    for c in range(NCHUNK):
      for g in range(NGRP):
        sl = pl.ds(c * CHUNK + g * 16, 16)
        bs_v[sl] = ubv[sl] + fbv[sl]

    acc_v[:] = (accs[0] + accs[1]) + (accs[2] + accs[3])
    pltpu.sync_copy(acc_v, part_h.at[pl.ds(wid * EMBED, EMBED)])
    pltpu.sync_copy(bs_v, bsum_h.at[pl.ds(base, BPW)])

  return k(uemb2, uidx, femb2, fidx, ubias, fbias)


def _tc_finish(partials, bsum):
  def body(p_ref, b_ref, o_ref):
    s = jnp.sum(p_ref[:])
    o_ref[:] = 1.0 / (1.0 + jnp.exp(-(b_ref[:] + s)))

  return pl.pallas_call(
      body,
      out_shape=jax.ShapeDtypeStruct((128, 128), jnp.float32),
  )(partials, bsum)


def kernel(inputs, user_embedding, user_bias, food_embedding, food_bias):
  uidx = inputs[:, 0].astype(jnp.int32).reshape(NW, NCHUNK, CHUNK)
  fidx = inputs[:, -1].astype(jnp.int32).reshape(NW, NCHUNK, CHUNK)
  uemb2 = user_embedding.reshape(125000, 128)
  femb2 = food_embedding.reshape(125000, 128)
  part, bsum = _sc_gather_partial(
      uemb2, uidx, femb2, fidx,
      user_bias.reshape(-1), food_bias.reshape(-1))
  out = _tc_finish(part.reshape(4, 128), bsum.reshape(128, 128))
  return out.reshape(B, 1)
